# Initial kernel scaffold; baseline (speedup 1.0000x reference)
import jax
import jax.numpy as jnp
from jax.experimental import pallas as pl


def kernel(pos):
    raise NotImplementedError("write your pallas kernel here")


# speedup vs baseline: 4.4199x; 4.4199x over previous
import functools

import jax
import jax.numpy as jnp
from jax.experimental import pallas as pl
from jax.experimental.pallas import tpu as pltpu

K = 32
N = 8192
ROWS = 256


def _knn_body(q_ref, k_ref, out_ref):
    q = q_ref[0]
    kk = k_ref[0]
    qsq = jnp.sum(q * q, axis=0)[:, None]
    ksq = jnp.sum(kk * kk, axis=0)[None, :]
    dot = jax.lax.dot_general(q, kk, (((0,), (0,)), ((), ())),
                              preferred_element_type=jnp.float32)
    d = qsq - 2.0 * dot + ksq

    col = jax.lax.broadcasted_iota(jnp.int32, (ROWS, N), 1)
    out_col = jax.lax.broadcasted_iota(jnp.int32, (ROWS, K), 1)
    big = jnp.int32(N)

    def step(j, carry):
        d, out = carry
        m = jnp.min(d, axis=1, keepdims=True)
        sel = d == m
        idx = jnp.min(jnp.where(sel, col, big), axis=1, keepdims=True)
        out = jnp.where(out_col == j, idx, out)
        d = jnp.where(col == idx, jnp.inf, d)
        return d, out

    _, out = jax.lax.fori_loop(0, K, step,
                               (d, jnp.zeros((ROWS, K), jnp.int32)))
    out_ref[0] = out


@jax.jit
def kernel(pos):
    B = pos.shape[0]
    grid = (B, N // ROWS)
    ids = pl.pallas_call(
        _knn_body,
        grid=grid,
        in_specs=[
            pl.BlockSpec((1, 3, ROWS), lambda b, i: (b, 0, i)),
            pl.BlockSpec((1, 3, N), lambda b, i: (b, 0, 0)),
        ],
        out_specs=pl.BlockSpec((1, ROWS, K), lambda b, i: (b, i, 0)),
        out_shape=jax.ShapeDtypeStruct((B, N, K), jnp.int32),
    )(pos, pos)
    return (pos, ids.astype(jnp.int64))
